# baseline (speedup 1.0000x reference)
import functools

import jax
import jax.numpy as jnp
from jax import lax
from jax.experimental import pallas as pl
from jax.experimental.pallas import tpu as pltpu
from jax.experimental.pallas import tpu_sc as plsc

NC = 2
NS = 16
NW = NC * NS

BB = 128
KG = 2
L16 = 16


def _make_emb(B0, B1, D):
    assert B0 == BB * NW and D % 8 == 0 and B1 % KG == 0
    TD = D // 8

    mesh = plsc.VectorSubcoreMesh(core_axis_name="c", subcore_axis_name="s")

    @functools.partial(
        pl.kernel,
        out_type=jax.ShapeDtypeStruct((B1, TD, NW, 8, BB), jnp.float32),
        mesh=mesh,
        scratch_types=[
            pltpu.VMEM((B1, BB), jnp.int32),
            pltpu.VMEM((KG, BB, D), jnp.float32),
            pltpu.VMEM((2, TD, 8, BB), jnp.float32),
        ]
        + [pltpu.SemaphoreType.DMA] * KG
        + [pltpu.SemaphoreType.DMA] * 2,
        compiler_params=pltpu.CompilerParams(
            use_tc_tiling_on_sc=False, needs_layout_passes=False
        ),
    )
    def emb(idx_hbm, table_hbm, out_hbm, idx_v, rows_v, slab_v, *sems):
        gsems, ssems = sems[:KG], sems[KG:]
        wid = lax.axis_index("s") * NC + lax.axis_index("c")
        pltpu.sync_copy(idx_hbm.at[:, pl.ds(wid * BB, BB)], idx_v)

        iota = lax.iota(jnp.int32, L16)
        rids = [iota + (g * L16) for g in range(BB // L16)]
        colids = [jnp.full((L16,), d, jnp.int32) for d in range(D)]

        def fire_gather(t, k):
            pltpu.async_copy(table_hbm.at[idx_v.at[t]], rows_v.at[k], gsems[k])

        def wait_gather(t, k):
            pltpu.make_async_copy(
                table_hbm.at[idx_v.at[t]], rows_v.at[k], gsems[k]
            ).wait()

        def fire_store(t, s):
            for td in range(TD):
                pltpu.async_copy(
                    slab_v.at[s, td], out_hbm.at[t, td, wid], ssems[s]
                )

        def wait_store(t, s):
            for td in range(TD):
                pltpu.make_async_copy(
                    slab_v.at[s, td], out_hbm.at[t, td, wid], ssems[s]
                ).wait()

        def transpose(k, s):
            for g in range(BB // L16):
                for d in range(D):
                    val = plsc.load_gather(rows_v.at[k], [rids[g], colids[d]])
                    slab_v[s, d // 8, d % 8, pl.ds(g * L16, L16)] = val

        def body(t, k, first, last):
            wait_gather(t, k)
            s = k
            if not first:
                wait_store(t - KG, s)
            transpose(k, s)
            fire_store(t, s)
            if not last:
                fire_gather(t + KG, k)

        for k in range(KG):
            fire_gather(k, k)
        for k in range(KG):
            body(k, k, first=True, last=False)

        def step(g, carry):
            t0 = KG + g * KG
            for k in range(KG):
                body(t0 + k, k, first=False, last=False)
            return carry

        n_mid = (B1 - 2 * KG) // KG
        lax.fori_loop(0, n_mid, step, 0)

        t0 = B1 - KG
        for k in range(KG):
            body(t0 + k, k, first=False, last=True)
        for k in range(KG):
            wait_store(B1 - KG + k, k)

    return emb


def kernel(H, table):
    B0, B1 = H.shape
    D = table.shape[1]
    Ht = H.T.astype(jnp.int32)
    out = _make_emb(B0, B1, D)(Ht, table)
    return out.transpose((2, 4, 0, 1, 3)).reshape(B0, B1, D)


# speedup vs baseline: 1.2222x; 1.2222x over previous
import functools

import jax
import jax.numpy as jnp
from jax import lax
from jax.experimental import pallas as pl
from jax.experimental.pallas import tpu as pltpu
from jax.experimental.pallas import tpu_sc as plsc

NC = 2
NS = 16
NW = NC * NS

CHUNK = 128
KBUF = 8


def _make_emb(B, D):
    n_ch = B // (NW * CHUNK)
    assert B == NW * n_ch * CHUNK and n_ch % KBUF == 0

    mesh = plsc.VectorSubcoreMesh(core_axis_name="c", subcore_axis_name="s")

    @functools.partial(
        pl.kernel,
        out_type=jax.ShapeDtypeStruct((B, D), jnp.float32),
        mesh=mesh,
        scratch_types=[
            pltpu.VMEM((n_ch, CHUNK), jnp.int32),
            pltpu.VMEM((KBUF, CHUNK, D), jnp.float32),
        ] + [pltpu.SemaphoreType.DMA] * KBUF,
        compiler_params=pltpu.CompilerParams(use_tc_tiling_on_sc=False),
    )
    def emb(idx_hbm, table_hbm, out_hbm, idx_v, rows_v, *gsems):
        wid = lax.axis_index("s") * NC + lax.axis_index("c")
        pltpu.sync_copy(idx_hbm.at[pl.ds(wid * n_ch, n_ch)], idx_v)
        base = wid * n_ch * CHUNK

        def gather(j, b):
            pltpu.async_copy(table_hbm.at[idx_v.at[j]], rows_v.at[b], gsems[b])

        def drain_store(j, b):
            pltpu.make_async_copy(
                table_hbm.at[idx_v.at[j]], rows_v.at[b], gsems[b]
            ).wait()
            pltpu.sync_copy(rows_v.at[b], out_hbm.at[pl.ds(base + j * CHUNK, CHUNK)])

        for b in range(KBUF):
            gather(b, b)

        def step(jj, carry):
            j0 = jj * KBUF
            for b in range(KBUF):
                drain_store(j0 + b, b)
                gather(j0 + b + KBUF, b)
            return carry

        lax.fori_loop(0, n_ch // KBUF - 1, step, 0)

        j0 = n_ch - KBUF
        for b in range(KBUF):
            drain_store(j0 + b, b)

    return emb


def kernel(H, table):
    B0, B1 = H.shape
    D = table.shape[1]
    B = B0 * B1
    idx = H.reshape(-1).astype(jnp.int32).reshape(B // CHUNK, CHUNK)
    out = _make_emb(B, D)(idx, table)
    return out.reshape(B0, B1, D)
